# Initial kernel scaffold; baseline (speedup 1.0000x reference)
import jax
import jax.numpy as jnp
from jax.experimental import pallas as pl


def kernel(x, edge_index, W1, b1, gamma, beta, W2, b2):
    raise NotImplementedError("write your pallas kernel here")


# speedup vs baseline: 10.1512x; 10.1512x over previous
import functools

import jax
import jax.numpy as jnp
from jax import lax
from jax.experimental import pallas as pl
from jax.experimental.pallas import tpu as pltpu
from jax.experimental.pallas import tpu_sc as plsc

_N = 10000
_E = 320000
_D = 128
_H = 256
_L = 4
_EPS = 1e-7

_NSC = 2
_NT = 16
_EPT = _E // _NT
_CH = 80
_NCH = _EPT // _CH
_NSB = 5
_BN = _NCH // _NSB
_RPT = 632
_NP = _RPT * _NT


def _sc_body(t_hbm, srcx_hbm, dst_hbm, o_hbm,
             src_v, dst_v, rows_v, accum_sh, sem):
    c = lax.axis_index("c")
    s = lax.axis_index("s")

    zero = jnp.zeros((16,), jnp.float32)

    def _zrow(i, carry):
        for j in range(_D // 16):
            rows_v[i, pl.ds(j * 16, 16)] = zero
        return carry

    lax.fori_loop(0, _CH, _zrow, 0)
    for r in range(7):
        pltpu.sync_copy(
            rows_v,
            accum_sh.at[pl.ds(pl.multiple_of(s * _RPT + r * _CH, 8), _CH)])
    pltpu.sync_copy(
        rows_v.at[pl.ds(0, _RPT - 7 * _CH)],
        accum_sh.at[pl.ds(pl.multiple_of(s * _RPT + 7 * _CH, 8),
                          _RPT - 7 * _CH)])
    plsc.subcore_barrier()

    def _chunk(k, carry):
        pltpu.async_copy(t_hbm.at[src_v.at[k]], rows_v, sem).wait()
        pltpu.sync_copy(rows_v, accum_sh.at[dst_v.at[k]], add=True)
        return carry

    def _superblock(b, carry):
        pltpu.sync_copy(srcx_hbm.at[c, s, b], src_v)
        pltpu.sync_copy(dst_hbm.at[s, b], dst_v)
        lax.fori_loop(0, _BN, _chunk, 0)
        return carry

    lax.fori_loop(0, _NSB, _superblock, 0)
    plsc.subcore_barrier()

    pltpu.sync_copy(
        accum_sh.at[pl.ds(pl.multiple_of(s * _RPT, 8), _RPT)],
        o_hbm.at[pl.ds(pl.multiple_of(c * _NP + s * _RPT, 8), _RPT)])


@functools.lru_cache(maxsize=None)
def _sc_scatter():
    return pl.kernel(
        _sc_body,
        out_type=jax.ShapeDtypeStruct((2 * _NP, _D), jnp.float32),
        mesh=plsc.VectorSubcoreMesh(core_axis_name="c", subcore_axis_name="s"),
        scratch_types=[
            pltpu.VMEM((_BN, _CH), jnp.int32),
            pltpu.VMEM((_BN, _CH), jnp.int32),
            pltpu.VMEM((_CH, _D), jnp.float32),
            pltpu.VMEM_SHARED((_NP, _D), jnp.float32),
            pltpu.SemaphoreType.DMA,
        ],
    )


def _tables(h, t_ref):
    v = jnp.maximum(h, 0.0) + _EPS
    cmax = jnp.max(v, axis=0, keepdims=True)
    q = jnp.exp(v - cmax)
    t_ref[0:_N, :] = q * v
    t_ref[_N:2 * _N, :] = q


def _tc_pre_body(x_ref, t_ref):
    _tables(x_ref[...], t_ref)


def _tc_layer_body(h_ref, o_ref, w1_ref, b1_ref, g_ref, be_ref, w2_ref,
                   b2_ref, *out_refs, last):
    num = o_ref[0:_N, :]
    den = o_ref[_NP:_NP + _N, :]
    agg = num / (den + 1e-30)
    hin = agg + h_ref[...]
    hm = jnp.dot(hin, w1_ref[...], preferred_element_type=jnp.float32,
                 precision=jax.lax.Precision.HIGHEST)
    hm = hm + b1_ref[...]
    mu = jnp.mean(hm, axis=0, keepdims=True)
    var = jnp.mean((hm - mu) ** 2, axis=0, keepdims=True)
    hb = (hm - mu) / jnp.sqrt(var + 1e-5) * g_ref[...] + be_ref[...]
    hb = jnp.maximum(hb, 0.0)
    h2 = jnp.dot(hb, w2_ref[...], preferred_element_type=jnp.float32,
                 precision=jax.lax.Precision.HIGHEST)
    h2 = h2 + b2_ref[...]
    if last:
        out_refs[0][...] = h2
    else:
        hh = h2 * jnp.tanh(jax.nn.softplus(h2))
        out_refs[0][...] = hh
        _tables(hh, out_refs[1])


_tc_pre = pl.pallas_call(
    _tc_pre_body,
    out_shape=jax.ShapeDtypeStruct((2 * _N, _D), jnp.float32),
)

_tc_mid = pl.pallas_call(
    functools.partial(_tc_layer_body, last=False),
    out_shape=[
        jax.ShapeDtypeStruct((_N, _D), jnp.float32),
        jax.ShapeDtypeStruct((2 * _N, _D), jnp.float32),
    ],
)

_OSHAPE = jax.ShapeDtypeStruct((2 * _NP, _D), jnp.float32)

_tc_fin = pl.pallas_call(
    functools.partial(_tc_layer_body, last=True),
    out_shape=jax.ShapeDtypeStruct((_N, _D), jnp.float32),
)


def kernel(x, edge_index, W1, b1, gamma, beta, W2, b2):
    src = edge_index[0]
    dst = edge_index[1]
    srcx = jnp.stack([src, src + _N]).reshape(_NSC, _NT, _NSB, _BN, _CH)
    dst_r = dst.reshape(_NT, _NSB, _BN, _CH)

    t = _tc_pre(x)
    h = x
    for i in range(_L):
        o = _sc_scatter()(t, srcx, dst_r)
        w1 = W1[i]
        b1i = b1[i].reshape(1, _H)
        g = gamma[i].reshape(1, _H)
        be = beta[i].reshape(1, _H)
        w2 = W2[i]
        b2i = b2[i].reshape(1, _D)
        if i < _L - 1:
            h, t = _tc_mid(h, o, w1, b1i, g, be, w2, b2i)
        else:
            h = _tc_fin(h, o, w1, b1i, g, be, w2, b2i)
    return h


# speedup vs baseline: 10.7365x; 1.0577x over previous
import functools

import jax
import jax.numpy as jnp
from jax import lax
from jax.experimental import pallas as pl
from jax.experimental.pallas import tpu as pltpu
from jax.experimental.pallas import tpu_sc as plsc

_N = 10000
_E = 320000
_D = 128
_H = 256
_L = 4
_EPS = 1e-7

_NSC = 2
_NT = 16
_EPT = _E // _NT
_CH = 80
_NCH = _EPT // _CH
_NSB = 5
_BN = _NCH // _NSB
_RPT = 632
_NP = _RPT * _NT


def _sc_body(t_hbm, srcx_hbm, dst_hbm, o_hbm,
             src_v, dst_v, rows_v, accum_sh, sem):
    c = lax.axis_index("c")
    s = lax.axis_index("s")

    zero = jnp.zeros((16,), jnp.float32)

    def _zrow(i, carry):
        for j in range(_D // 16):
            rows_v[i, pl.ds(j * 16, 16)] = zero
        return carry

    lax.fori_loop(0, _CH, _zrow, 0)
    for r in range(7):
        pltpu.sync_copy(
            rows_v,
            accum_sh.at[pl.ds(pl.multiple_of(s * _RPT + r * _CH, 8), _CH)])
    pltpu.sync_copy(
        rows_v.at[pl.ds(0, _RPT - 7 * _CH)],
        accum_sh.at[pl.ds(pl.multiple_of(s * _RPT + 7 * _CH, 8),
                          _RPT - 7 * _CH)])
    plsc.subcore_barrier()

    def _chunk(k, carry):
        pltpu.async_copy(t_hbm.at[src_v.at[k]], rows_v, sem).wait()
        pltpu.sync_copy(rows_v, accum_sh.at[dst_v.at[k]], add=True)
        return carry

    def _superblock(b, carry):
        pltpu.sync_copy(srcx_hbm.at[c, s, b], src_v)
        pltpu.sync_copy(dst_hbm.at[s, b], dst_v)
        lax.fori_loop(0, _BN, _chunk, 0)
        return carry

    lax.fori_loop(0, _NSB, _superblock, 0)
    plsc.subcore_barrier()

    pltpu.sync_copy(
        accum_sh.at[pl.ds(pl.multiple_of(s * _RPT, 8), _RPT)],
        o_hbm.at[pl.ds(pl.multiple_of(c * _NP + s * _RPT, 8), _RPT)])


@functools.lru_cache(maxsize=None)
def _sc_scatter():
    return pl.kernel(
        _sc_body,
        out_type=jax.ShapeDtypeStruct((2 * _NP, _D), jnp.float32),
        mesh=plsc.VectorSubcoreMesh(core_axis_name="c", subcore_axis_name="s"),
        scratch_types=[
            pltpu.VMEM((_BN, _CH), jnp.int32),
            pltpu.VMEM((_BN, _CH), jnp.int32),
            pltpu.VMEM((_CH, _D), jnp.float32),
            pltpu.VMEM_SHARED((_NP, _D), jnp.float32),
            pltpu.SemaphoreType.DMA,
        ],
    )


def _tables(h, t_ref):
    v = jnp.maximum(h, 0.0) + _EPS
    cmax = jnp.max(v, axis=0, keepdims=True)
    q = jnp.exp(v - cmax)
    t_ref[0:_N, :] = q * v
    t_ref[_N:2 * _N, :] = q


def _tc_pre_body(x_ref, t_ref):
    _tables(x_ref[...], t_ref)


def _tc_layer_body(h_ref, o_ref, w1_ref, b1_ref, g_ref, be_ref, w2_ref,
                   b2_ref, *out_refs, last):
    num = o_ref[0:_N, :]
    den = o_ref[_NP:_NP + _N, :]
    agg = num / (den + 1e-30)
    hin = agg + h_ref[...]
    hm = jnp.dot(hin, w1_ref[...], preferred_element_type=jnp.float32)
    hm = hm + b1_ref[...]
    mu = jnp.mean(hm, axis=0, keepdims=True)
    var = jnp.mean((hm - mu) ** 2, axis=0, keepdims=True)
    hb = (hm - mu) / jnp.sqrt(var + 1e-5) * g_ref[...] + be_ref[...]
    hb = jnp.maximum(hb, 0.0)
    h2 = jnp.dot(hb, w2_ref[...], preferred_element_type=jnp.float32)
    h2 = h2 + b2_ref[...]
    if last:
        out_refs[0][...] = h2
    else:
        hh = h2 * jnp.tanh(jax.nn.softplus(h2))
        out_refs[0][...] = hh
        _tables(hh, out_refs[1])


_tc_pre = pl.pallas_call(
    _tc_pre_body,
    out_shape=jax.ShapeDtypeStruct((2 * _N, _D), jnp.float32),
)

_tc_mid = pl.pallas_call(
    functools.partial(_tc_layer_body, last=False),
    out_shape=[
        jax.ShapeDtypeStruct((_N, _D), jnp.float32),
        jax.ShapeDtypeStruct((2 * _N, _D), jnp.float32),
    ],
)

_OSHAPE = jax.ShapeDtypeStruct((2 * _NP, _D), jnp.float32)

_tc_fin = pl.pallas_call(
    functools.partial(_tc_layer_body, last=True),
    out_shape=jax.ShapeDtypeStruct((_N, _D), jnp.float32),
)


def kernel(x, edge_index, W1, b1, gamma, beta, W2, b2):
    src = edge_index[0]
    dst = edge_index[1]
    srcx = jnp.stack([src, src + _N]).reshape(_NSC, _NT, _NSB, _BN, _CH)
    dst_r = dst.reshape(_NT, _NSB, _BN, _CH)

    t = _tc_pre(x)
    h = x
    for i in range(_L):
        o = _sc_scatter()(t, srcx, dst_r)
        w1 = W1[i]
        b1i = b1[i].reshape(1, _H)
        g = gamma[i].reshape(1, _H)
        be = beta[i].reshape(1, _H)
        w2 = W2[i]
        b2i = b2[i].reshape(1, _D)
        if i < _L - 1:
            h, t = _tc_mid(h, o, w1, b1i, g, be, w2, b2i)
        else:
            h = _tc_fin(h, o, w1, b1i, g, be, w2, b2i)
    return h


# speedup vs baseline: 17.7857x; 1.6566x over previous
import functools

import jax
import jax.numpy as jnp
from jax import lax
from jax.experimental import pallas as pl
from jax.experimental.pallas import tpu as pltpu
from jax.experimental.pallas import tpu_sc as plsc

_N = 10000
_E = 320000
_D = 128
_H = 256
_L = 4
_EPS = 1e-7

_NSC = 2
_NT = 16
_EPT = _E // _NT
_CH = 80
_NCH = _EPT // _CH
_NSB = 5
_BN = _NCH // _NSB
_RPT = 632
_NP = _RPT * _NT


def _sc_body(t_hbm, srcx_hbm, dst_hbm, o_hbm,
             src_v, dst_v, rows_v, accum_sh, sem0, sem1):
    c = lax.axis_index("c")
    s = lax.axis_index("s")
    r0 = rows_v.at[pl.ds(0, _CH)]
    r1 = rows_v.at[pl.ds(_CH, _CH)]

    zero = jnp.zeros((16,), jnp.float32)

    def _zrow(i, carry):
        for j in range(_D // 16):
            rows_v[i, pl.ds(j * 16, 16)] = zero
        return carry

    lax.fori_loop(0, 2 * _CH, _zrow, 0)
    nz = _RPT // (2 * _CH)
    for r in range(nz):
        pltpu.sync_copy(
            rows_v,
            accum_sh.at[pl.ds(pl.multiple_of(s * _RPT + r * 2 * _CH, 8),
                              2 * _CH)])
    rem = _RPT - nz * 2 * _CH
    pltpu.sync_copy(
        rows_v.at[pl.ds(0, rem)],
        accum_sh.at[pl.ds(pl.multiple_of(s * _RPT + nz * 2 * _CH, 8), rem)])
    plsc.subcore_barrier()

    def _superblock(b, carry):
        pltpu.sync_copy(srcx_hbm.at[c, s, b], src_v)
        pltpu.sync_copy(dst_hbm.at[s, b], dst_v)
        pltpu.async_copy(t_hbm.at[src_v.at[0]], r0, sem0)
        pltpu.async_copy(t_hbm.at[src_v.at[1]], r1, sem1)

        def _pair(p, carry2):
            k0 = 2 * p
            pltpu.make_async_copy(t_hbm.at[src_v.at[k0]], r0, sem0).wait()
            pltpu.sync_copy(r0, accum_sh.at[dst_v.at[k0]], add=True)

            @pl.when(p < _BN // 2 - 1)
            def _():
                pltpu.async_copy(t_hbm.at[src_v.at[k0 + 2]], r0, sem0)

            pltpu.make_async_copy(t_hbm.at[src_v.at[k0 + 1]], r1, sem1).wait()
            pltpu.sync_copy(r1, accum_sh.at[dst_v.at[k0 + 1]], add=True)

            @pl.when(p < _BN // 2 - 1)
            def _():
                pltpu.async_copy(t_hbm.at[src_v.at[k0 + 3]], r1, sem1)

            return carry2

        lax.fori_loop(0, _BN // 2, _pair, 0)
        return carry

    lax.fori_loop(0, _NSB, _superblock, 0)
    plsc.subcore_barrier()

    pltpu.sync_copy(
        accum_sh.at[pl.ds(pl.multiple_of(s * _RPT, 8), _RPT)],
        o_hbm.at[pl.ds(pl.multiple_of(c * _NP + s * _RPT, 8), _RPT)])


@functools.lru_cache(maxsize=None)
def _sc_scatter():
    return pl.kernel(
        _sc_body,
        out_type=jax.ShapeDtypeStruct((2 * _NP, _D), jnp.float32),
        mesh=plsc.VectorSubcoreMesh(core_axis_name="c", subcore_axis_name="s"),
        scratch_types=[
            pltpu.VMEM((_BN, _CH), jnp.int32),
            pltpu.VMEM((_BN, _CH), jnp.int32),
            pltpu.VMEM((2 * _CH, _D), jnp.float32),
            pltpu.VMEM_SHARED((_NP, _D), jnp.float32),
            pltpu.SemaphoreType.DMA,
            pltpu.SemaphoreType.DMA,
        ],
    )


def _tables(h, t_ref):
    v = jnp.maximum(h, 0.0) + _EPS
    cmax = jnp.max(v, axis=0, keepdims=True)
    q = jnp.exp(v - cmax)
    t_ref[0:_N, :] = q * v
    t_ref[_N:2 * _N, :] = q


def _tc_pre_body(x_ref, t_ref):
    _tables(x_ref[...], t_ref)


def _tc_layer_body(h_ref, o_ref, w1_ref, b1_ref, g_ref, be_ref, w2_ref,
                   b2_ref, *out_refs, last):
    num = o_ref[0:_N, :]
    den = o_ref[_NP:_NP + _N, :]
    agg = num / (den + 1e-30)
    hin = agg + h_ref[...]
    hm = jnp.dot(hin, w1_ref[...], preferred_element_type=jnp.float32)
    hm = hm + b1_ref[...]
    mu = jnp.mean(hm, axis=0, keepdims=True)
    var = jnp.mean((hm - mu) ** 2, axis=0, keepdims=True)
    hb = (hm - mu) / jnp.sqrt(var + 1e-5) * g_ref[...] + be_ref[...]
    hb = jnp.maximum(hb, 0.0)
    h2 = jnp.dot(hb, w2_ref[...], preferred_element_type=jnp.float32)
    h2 = h2 + b2_ref[...]
    if last:
        out_refs[0][...] = h2
    else:
        hh = h2 * jnp.tanh(jax.nn.softplus(h2))
        out_refs[0][...] = hh
        _tables(hh, out_refs[1])


_tc_pre = pl.pallas_call(
    _tc_pre_body,
    out_shape=jax.ShapeDtypeStruct((2 * _N, _D), jnp.float32),
)

_tc_mid = pl.pallas_call(
    functools.partial(_tc_layer_body, last=False),
    out_shape=[
        jax.ShapeDtypeStruct((_N, _D), jnp.float32),
        jax.ShapeDtypeStruct((2 * _N, _D), jnp.float32),
    ],
)

_OSHAPE = jax.ShapeDtypeStruct((2 * _NP, _D), jnp.float32)

_tc_fin = pl.pallas_call(
    functools.partial(_tc_layer_body, last=True),
    out_shape=jax.ShapeDtypeStruct((_N, _D), jnp.float32),
)


def kernel(x, edge_index, W1, b1, gamma, beta, W2, b2):
    src = edge_index[0]
    dst = edge_index[1]
    srcx = jnp.stack([src, src + _N]).reshape(_NSC, _NT, _NSB, _BN, _CH)
    dst_r = dst.reshape(_NT, _NSB, _BN, _CH)

    t = _tc_pre(x)
    h = x
    for i in range(_L):
        o = _sc_scatter()(t, srcx, dst_r)
        w1 = W1[i]
        b1i = b1[i].reshape(1, _H)
        g = gamma[i].reshape(1, _H)
        be = beta[i].reshape(1, _H)
        w2 = W2[i]
        b2i = b2[i].reshape(1, _D)
        if i < _L - 1:
            h, t = _tc_mid(h, o, w1, b1i, g, be, w2, b2i)
        else:
            h = _tc_fin(h, o, w1, b1i, g, be, w2, b2i)
    return h


# speedup vs baseline: 21.1784x; 1.1908x over previous
import functools

import jax
import jax.numpy as jnp
from jax import lax
from jax.experimental import pallas as pl
from jax.experimental.pallas import tpu as pltpu
from jax.experimental.pallas import tpu_sc as plsc

_N = 10000
_E = 320000
_D = 128
_H = 256
_L = 4
_EPS = 1e-7

_NSC = 2
_NT = 16
_EPT = _E // _NT
_CH = 80
_NCH = _EPT // _CH
_NSB = 5
_BN = _NCH // _NSB
_RPT = 632
_NP = _RPT * _NT


_NBUF = 3


def _sc_body(t_hbm, srcx_hbm, dst_hbm, o_hbm,
             src_v, dst_v, rows_v, accum_sh, *sems):
    c = lax.axis_index("c")
    s = lax.axis_index("s")
    rbufs = [rows_v.at[pl.ds(j * _CH, _CH)] for j in range(_NBUF)]

    zero = jnp.zeros((16,), jnp.float32)

    def _zrow(i, carry):
        for j in range(_D // 16):
            rows_v[i, pl.ds(j * 16, 16)] = zero
        return carry

    lax.fori_loop(0, _NBUF * _CH, _zrow, 0)
    nz = _RPT // (_NBUF * _CH)
    for r in range(nz):
        pltpu.sync_copy(
            rows_v,
            accum_sh.at[pl.ds(pl.multiple_of(s * _RPT + r * _NBUF * _CH, 8),
                              _NBUF * _CH)])
    rem = _RPT - nz * _NBUF * _CH
    if rem:
        pltpu.sync_copy(
            rows_v.at[pl.ds(0, rem)],
            accum_sh.at[pl.ds(pl.multiple_of(s * _RPT + nz * _NBUF * _CH, 8),
                              rem)])
    plsc.subcore_barrier()

    ngrp = _BN // _NBUF
    ntail = _BN - ngrp * _NBUF

    def _superblock(b, carry):
        pltpu.sync_copy(srcx_hbm.at[c, s, b], src_v)
        pltpu.sync_copy(dst_hbm.at[s, b], dst_v)
        for j in range(_NBUF):
            pltpu.async_copy(t_hbm.at[src_v.at[j]], rbufs[j], sems[j])

        def _grp(g, carry2):
            k0 = _NBUF * g
            for j in range(_NBUF):
                pltpu.make_async_copy(
                    t_hbm.at[src_v.at[k0 + j]], rbufs[j], sems[j]).wait()
                pltpu.sync_copy(
                    rbufs[j], accum_sh.at[dst_v.at[k0 + j]], add=True)

                @pl.when(k0 + _NBUF + j < _BN)
                def _():
                    pltpu.async_copy(
                        t_hbm.at[src_v.at[k0 + _NBUF + j]], rbufs[j], sems[j])

            return carry2

        lax.fori_loop(0, ngrp, _grp, 0)
        for j in range(ntail):
            k = ngrp * _NBUF + j
            pltpu.make_async_copy(
                t_hbm.at[src_v.at[k]], rbufs[j], sems[j]).wait()
            pltpu.sync_copy(rbufs[j], accum_sh.at[dst_v.at[k]], add=True)
        return carry

    lax.fori_loop(0, _NSB, _superblock, 0)
    plsc.subcore_barrier()

    pltpu.sync_copy(
        accum_sh.at[pl.ds(pl.multiple_of(s * _RPT, 8), _RPT)],
        o_hbm.at[pl.ds(pl.multiple_of(c * _NP + s * _RPT, 8), _RPT)])


@functools.lru_cache(maxsize=None)
def _sc_scatter():
    return pl.kernel(
        _sc_body,
        out_type=jax.ShapeDtypeStruct((2 * _NP, _D), jnp.float32),
        mesh=plsc.VectorSubcoreMesh(core_axis_name="c", subcore_axis_name="s"),
        scratch_types=[
            pltpu.VMEM((_BN, _CH), jnp.int32),
            pltpu.VMEM((_BN, _CH), jnp.int32),
            pltpu.VMEM((_NBUF * _CH, _D), jnp.float32),
            pltpu.VMEM_SHARED((_NP, _D), jnp.float32),
        ] + [pltpu.SemaphoreType.DMA] * _NBUF,
    )


def _tables(h, t_ref):
    v = jnp.maximum(h, 0.0) + _EPS
    cmax = jnp.max(v, axis=0, keepdims=True)
    q = jnp.exp(v - cmax)
    t_ref[0:_N, :] = q * v
    t_ref[_N:2 * _N, :] = q


def _tc_pre_body(x_ref, t_ref):
    _tables(x_ref[...], t_ref)


def _tc_layer_body(h_ref, o_ref, w1_ref, b1_ref, g_ref, be_ref, w2_ref,
                   b2_ref, *out_refs, last):
    num = o_ref[0:_N, :]
    den = o_ref[_NP:_NP + _N, :]
    agg = num / (den + 1e-30)
    hin = agg + h_ref[...]
    hm = jnp.dot(hin, w1_ref[...], preferred_element_type=jnp.float32)
    hm = hm + b1_ref[...]
    mu = jnp.mean(hm, axis=0, keepdims=True)
    var = jnp.mean((hm - mu) ** 2, axis=0, keepdims=True)
    hb = (hm - mu) / jnp.sqrt(var + 1e-5) * g_ref[...] + be_ref[...]
    hb = jnp.maximum(hb, 0.0)
    h2 = jnp.dot(hb, w2_ref[...], preferred_element_type=jnp.float32)
    h2 = h2 + b2_ref[...]
    if last:
        out_refs[0][...] = h2
    else:
        hh = h2 * jnp.tanh(jax.nn.softplus(h2))
        out_refs[0][...] = hh
        _tables(hh, out_refs[1])


_tc_pre = pl.pallas_call(
    _tc_pre_body,
    out_shape=jax.ShapeDtypeStruct((2 * _N, _D), jnp.float32),
)

_tc_mid = pl.pallas_call(
    functools.partial(_tc_layer_body, last=False),
    out_shape=[
        jax.ShapeDtypeStruct((_N, _D), jnp.float32),
        jax.ShapeDtypeStruct((2 * _N, _D), jnp.float32),
    ],
)

_OSHAPE = jax.ShapeDtypeStruct((2 * _NP, _D), jnp.float32)

_tc_fin = pl.pallas_call(
    functools.partial(_tc_layer_body, last=True),
    out_shape=jax.ShapeDtypeStruct((_N, _D), jnp.float32),
)


def kernel(x, edge_index, W1, b1, gamma, beta, W2, b2):
    src = edge_index[0]
    dst = edge_index[1]
    srcx = jnp.stack([src, src + _N]).reshape(_NSC, _NT, _NSB, _BN, _CH)
    dst_r = dst.reshape(_NT, _NSB, _BN, _CH)

    t = _tc_pre(x)
    h = x
    for i in range(_L):
        o = _sc_scatter()(t, srcx, dst_r)
        w1 = W1[i]
        b1i = b1[i].reshape(1, _H)
        g = gamma[i].reshape(1, _H)
        be = beta[i].reshape(1, _H)
        w2 = W2[i]
        b2i = b2[i].reshape(1, _D)
        if i < _L - 1:
            h, t = _tc_mid(h, o, w1, b1i, g, be, w2, b2i)
        else:
            h = _tc_fin(h, o, w1, b1i, g, be, w2, b2i)
    return h
